# Initial kernel scaffold; baseline (speedup 1.0000x reference)
import jax
import jax.numpy as jnp
from jax.experimental import pallas as pl


def kernel(x, Wg, W1, b1, W2, b2):
    raise NotImplementedError("write your pallas kernel here")


# speedup vs baseline: 1.5546x; 1.5546x over previous
import jax
import jax.numpy as jnp
from jax.experimental import pallas as pl
from jax.experimental.pallas import tpu as pltpu

D_MODEL = 768
HIDDEN = 1536
N_EXPERTS = 8
N_TOKENS = 2048


def _moe_dense_body(x_ref, wg_ref, w1_ref, b1_ref, w2_ref, b2_ref,
                    out_ref, acc_ref, gates_ref):
    e = pl.program_id(0)

    @pl.when(e == 0)
    def _():
        x = x_ref[...]
        logits = jnp.dot(x, wg_ref[...], preferred_element_type=jnp.float32)
        m1 = jnp.max(logits, axis=-1, keepdims=True)
        eids = jax.lax.broadcasted_iota(jnp.int32, logits.shape, 1)
        i1 = jnp.argmax(logits, axis=-1)[:, None]
        masked = jnp.where(eids == i1, -jnp.inf, logits)
        m2 = jnp.max(masked, axis=-1, keepdims=True)
        i2 = jnp.argmax(masked, axis=-1)[:, None]
        t = jnp.exp(m2 - m1)
        g1 = 1.0 / (1.0 + t)
        g2 = 1.0 - g1
        gates = jnp.where(eids == i1, g1, jnp.where(eids == i2, g2, 0.0))
        gates_ref[...] = gates

    x = x_ref[...]
    h = jnp.dot(x, w1_ref[0], preferred_element_type=jnp.float32)
    h = jnp.maximum(h + b1_ref[0], 0.0)
    y = jnp.dot(h, w2_ref[0], preferred_element_type=jnp.float32)
    y = y + b2_ref[0]
    lane = jax.lax.broadcasted_iota(jnp.int32, (N_TOKENS, N_EXPERTS), 1)
    g = jnp.sum(jnp.where(lane == e, gates_ref[...], 0.0), axis=-1,
                keepdims=True)

    @pl.when(e == 0)
    def _():
        acc_ref[...] = g * y

    @pl.when(e > 0)
    def _():
        acc_ref[...] += g * y

    @pl.when(e == N_EXPERTS - 1)
    def _():
        out_ref[...] = acc_ref[...]


def kernel(x, Wg, W1, b1, W2, b2):
    return pl.pallas_call(
        _moe_dense_body,
        grid=(N_EXPERTS,),
        in_specs=[
            pl.BlockSpec((N_TOKENS, D_MODEL), lambda e: (0, 0)),
            pl.BlockSpec((D_MODEL, N_EXPERTS), lambda e: (0, 0)),
            pl.BlockSpec((1, D_MODEL, HIDDEN), lambda e: (e, 0, 0)),
            pl.BlockSpec((1, 1, HIDDEN), lambda e: (e, 0, 0)),
            pl.BlockSpec((1, HIDDEN, D_MODEL), lambda e: (e, 0, 0)),
            pl.BlockSpec((1, 1, D_MODEL), lambda e: (e, 0, 0)),
        ],
        out_specs=pl.BlockSpec((N_TOKENS, D_MODEL), lambda e: (0, 0)),
        out_shape=jax.ShapeDtypeStruct((N_TOKENS, D_MODEL), jnp.float32),
        scratch_shapes=[
            pltpu.VMEM((N_TOKENS, D_MODEL), jnp.float32),
            pltpu.VMEM((N_TOKENS, N_EXPERTS), jnp.float32),
        ],
    )(x, Wg, W1, b1[:, None, :], W2, b2[:, None, :])
